# baseline (speedup 1.0000x reference)
import functools

import jax
import jax.numpy as jnp
from jax import lax
from jax.experimental import pallas as pl
from jax.experimental.pallas import tpu as pltpu
from jax.experimental.pallas import tpu_sc as plsc

N = 10000
E = 320000
D = 128

NC = 2
NS = 16
NW = NC * NS

EPW = E // NW
CB = 128
NCHUNK = 80
GC = 8
NG = NCHUNK // GC
EPW_PAD = NCHUNK * CB

NA = N + 8

ROWS_PT = N // 10
CNT_PT = N // 10


def _matmul_body(x_ref, wa_ref, wb_ref, b_ref, a_out, b_out):
    xb = x_ref[...]
    a_out[...] = jnp.dot(xb, wa_ref[...], preferred_element_type=jnp.float32) + b_ref[...]
    b_out[...] = jnp.dot(xb, wb_ref[...], preferred_element_type=jnp.float32)


def _node_features(x_pad, wa, wb, b2d):
    return pl.pallas_call(
        _matmul_body,
        out_shape=(
            jax.ShapeDtypeStruct((NA, D), jnp.float32),
            jax.ShapeDtypeStruct((NA, D), jnp.float32),
        ),
    )(x_pad, wa, wb, b2d)


def _edge_body(a_hbm, b_hbm, dg_hbm, sg_hbm, p_hbm, cnt0_hbm, cnt1_hbm,
               acc, cnt_s, dg_v, sg_v, a0, b0, ones_v, zcnt,
               sem_a0, sem_b0, sem_s, sem_c):
    cid = lax.axis_index("c")
    sid = lax.axis_index("s")

    def _zero_a0(r, _):
        for k in range(8):
            a0[r, pl.ds(k * 16, 16)] = jnp.zeros((16,), jnp.float32)
        return ()
    lax.fori_loop(0, CB, _zero_a0, (), unroll=False)

    def _zero_zcnt(i, _):
        zcnt[pl.ds(i * 16, 16)] = jnp.zeros((16,), jnp.float32)
        return ()
    lax.fori_loop(0, 63, _zero_zcnt, (), unroll=False)

    ZT = ROWS_PT - (ROWS_PT // CB) * CB

    @pl.when(sid < 10)
    def _():
        for j in range(ROWS_PT // CB):
            base = sid * ROWS_PT + j * CB
            pltpu.sync_copy(a0, acc.at[pl.ds(base, CB)])
        pltpu.sync_copy(a0.at[pl.ds(0, ZT)],
                        acc.at[pl.ds(sid * ROWS_PT + (ROWS_PT // CB) * CB, ZT)])
        pltpu.sync_copy(zcnt.at[pl.ds(0, CNT_PT)],
                        cnt_s.at[pl.ds(sid * CNT_PT, CNT_PT)])

    @pl.when(sid == 10)
    def _():
        pltpu.sync_copy(a0.at[pl.ds(0, 8)], acc.at[pl.ds(N, 8)])
        pltpu.sync_copy(zcnt.at[pl.ds(0, 8)], cnt_s.at[pl.ds(N, 8)])

    for k in range(CB // 16):
        ones_v[pl.ds(k * 16, 16)] = jnp.ones((16,), jnp.float32)

    plsc.subcore_barrier()

    wid = cid * NS + sid

    def _chunk(k, _):
        idx_d = dg_v.at[0, k]
        idx_s = sg_v.at[0, k]
        ga = pltpu.async_copy(a_hbm.at[idx_d], a0, sem_a0)
        gb = pltpu.async_copy(b_hbm.at[idx_s], b0, sem_b0)
        ga.wait()
        gb.wait()

        def _row(e, _):
            for kk in range(8):
                sl = pl.ds(kk * 16, 16)
                v = a0[e, sl] + b0[e, sl]
                b0[e, sl] = jnp.maximum(v, 0.0)
            return ()
        lax.fori_loop(0, CB, _row, (), unroll=False)

        sc = pltpu.async_copy(b0, acc.at[idx_d], sem_s, add=True)
        cc = pltpu.async_copy(ones_v, cnt_s.at[idx_d], sem_c, add=True)
        sc.wait()
        cc.wait()
        return ()

    def _group(g, _):
        pltpu.sync_copy(dg_hbm.at[wid, pl.ds(g * GC, GC)], dg_v.at[0])
        pltpu.sync_copy(sg_hbm.at[wid, pl.ds(g * GC, GC)], sg_v.at[0])
        lax.fori_loop(0, GC, _chunk, (), unroll=False)
        return ()
    lax.fori_loop(0, NG, _group, (), unroll=False)

    plsc.subcore_barrier()

    @pl.when(sid < 10)
    def _():
        for j in range(ROWS_PT // CB):
            base = sid * ROWS_PT + j * CB
            pltpu.sync_copy(acc.at[pl.ds(base, CB)], b0)
            pltpu.sync_copy(b0, p_hbm.at[cid, pl.ds(base, CB)])
        tail = sid * ROWS_PT + (ROWS_PT // CB) * CB
        pltpu.sync_copy(acc.at[pl.ds(tail, ZT)], b0.at[pl.ds(0, ZT)])
        pltpu.sync_copy(b0.at[pl.ds(0, ZT)], p_hbm.at[cid, pl.ds(tail, ZT)])

        pltpu.sync_copy(cnt_s.at[pl.ds(sid * CNT_PT, CNT_PT)],
                        zcnt.at[pl.ds(0, CNT_PT)])

        @pl.when(cid == 0)
        def _():
            pltpu.sync_copy(zcnt.at[pl.ds(0, CNT_PT)],
                            cnt0_hbm.at[pl.ds(sid * CNT_PT, CNT_PT)])

        @pl.when(cid == 1)
        def _():
            pltpu.sync_copy(zcnt.at[pl.ds(0, CNT_PT)],
                            cnt1_hbm.at[pl.ds(sid * CNT_PT, CNT_PT)])


@functools.partial(
    pl.kernel,
    out_type=(
        jax.ShapeDtypeStruct((NC, N, D), jnp.float32),
        jax.ShapeDtypeStruct((N,), jnp.float32),
        jax.ShapeDtypeStruct((N,), jnp.float32),
    ),
    mesh=plsc.VectorSubcoreMesh(
        core_axis_name="c", subcore_axis_name="s", num_cores=NC, num_subcores=NS
    ),
    scratch_types=[
        pltpu.VMEM_SHARED((NA, D), jnp.float32),
        pltpu.VMEM_SHARED((NA,), jnp.float32),
        pltpu.VMEM((1, GC, CB), jnp.int32),
        pltpu.VMEM((1, GC, CB), jnp.int32),
        pltpu.VMEM((CB, D), jnp.float32),
        pltpu.VMEM((CB, D), jnp.float32),
        pltpu.VMEM((CB,), jnp.float32),
        pltpu.VMEM((1008,), jnp.float32),
        pltpu.SemaphoreType.DMA,
        pltpu.SemaphoreType.DMA,
        pltpu.SemaphoreType.DMA,
        pltpu.SemaphoreType.DMA,
    ],
)
def _edge_kernel(a_hbm, b_hbm, dg_hbm, sg_hbm, p_hbm,
                 cnt0_hbm, cnt1_hbm, *scratch):
    _edge_body(a_hbm, b_hbm, dg_hbm, sg_hbm, p_hbm,
               cnt0_hbm, cnt1_hbm, *scratch)


def _finalize_body(p_ref, c0_ref, c1_ref, o_ref):
    cnt = c0_ref[...] + c1_ref[...]
    inv = 1.0 / jnp.maximum(cnt, 1.0)
    o_ref[...] = (p_ref[0] + p_ref[1]) * inv[:, None]


def _finalize(p, cnt0, cnt1):
    return pl.pallas_call(
        _finalize_body,
        out_shape=jax.ShapeDtypeStruct((N, D), jnp.float32),
    )(p, cnt0, cnt1)


def _pad_edges(idx, pad_value):
    per_w = idx.reshape(NW, EPW)
    padded = jnp.pad(per_w, ((0, 0), (0, EPW_PAD - EPW)),
                     constant_values=pad_value)
    return padded.reshape(NW, NCHUNK, CB)


def kernel(x, edge_index, W, b):
    w1 = W[:, :D]
    w2 = W[:, D:]
    wa = (w1 - w2).T
    wb = w2.T
    b2d = b[None, :]
    x_pad = jnp.pad(x, ((0, NA - N), (0, 0)))
    a_nodes, b_nodes = _node_features(x_pad, wa, wb, b2d)
    src = edge_index[0]
    dst = edge_index[1]
    dst_idx = _pad_edges(dst, N)
    src_idx = _pad_edges(src, N)
    p, cnt0, cnt1 = _edge_kernel(a_nodes, b_nodes, dst_idx, src_idx)
    return _finalize(p, cnt0, cnt1)


# speedup vs baseline: 1.8079x; 1.8079x over previous
import functools

import jax
import jax.numpy as jnp
from jax import lax
from jax.experimental import pallas as pl
from jax.experimental.pallas import tpu as pltpu
from jax.experimental.pallas import tpu_sc as plsc

N = 10000
E = 320000
D = 128

NC = 2
NS = 16
NW = NC * NS

EPW = E // NW
CB = 80
NCHUNK = EPW // CB
GC = 8

ROWS_PT = N // 10
CNT_PT = N // 10


def _matmul_body(x_ref, wa_ref, wb_ref, b_ref, a_out, b_out):
    xb = x_ref[...]
    a_out[...] = jnp.dot(xb, wa_ref[...], preferred_element_type=jnp.float32) + b_ref[...]
    b_out[...] = jnp.dot(xb, wb_ref[...], preferred_element_type=jnp.float32)


def _node_features(x, wa, wb, b2d):
    return pl.pallas_call(
        _matmul_body,
        out_shape=(
            jax.ShapeDtypeStruct((N, D), jnp.float32),
            jax.ShapeDtypeStruct((N, D), jnp.float32),
        ),
    )(x, wa, wb, b2d)


def _edge_body(a_hbm, b_hbm, dst_hbm, src_hbm, p_hbm, cnt0_hbm, cnt1_hbm,
               acc, cnt_s, dst_v, src_v, arows, brows, m0, m1, ones_v, zcnt,
               sem_a, sem_b, sem_s0, sem_s1, sem_c0, sem_c1):
    cid = lax.axis_index("c")
    sid = lax.axis_index("s")

    def _zero_arows(r, _):
        for k in range(8):
            arows[r, pl.ds(k * 16, 16)] = jnp.zeros((16,), jnp.float32)
        return ()
    lax.fori_loop(0, CB, _zero_arows, (), unroll=False)

    def _zero_zcnt(i, _):
        zcnt[pl.ds(i * 16, 16)] = jnp.zeros((16,), jnp.float32)
        return ()
    lax.fori_loop(0, 63, _zero_zcnt, (), unroll=False)

    @pl.when(sid < 10)
    def _():
        for j in range(ROWS_PT // CB):
            base = sid * ROWS_PT + j * CB
            pltpu.sync_copy(arows, acc.at[pl.ds(base, CB)])
        pltpu.sync_copy(arows.at[pl.ds(0, 40)],
                        acc.at[pl.ds(sid * ROWS_PT + (ROWS_PT // CB) * CB, 40)])
        pltpu.sync_copy(zcnt.at[pl.ds(0, CNT_PT)],
                        cnt_s.at[pl.ds(sid * CNT_PT, CNT_PT)])

    for k in range(5):
        ones_v[pl.ds(k * 16, 16)] = jnp.ones((16,), jnp.float32)

    plsc.subcore_barrier()

    wid = cid * NS + sid

    mbuf = (m0, m1)
    ssem = (sem_s0, sem_s1)
    csem = (sem_c0, sem_c1)

    def _chunk(k, nk):
        i = k % 2
        idx_d = dst_v.at[k]
        idx_s = src_v.at[k]
        cp_a = pltpu.async_copy(a_hbm.at[idx_d], arows, sem_a)
        cp_b = pltpu.async_copy(b_hbm.at[idx_s], brows, sem_b)
        cp_a.wait()
        cp_b.wait()
        if k >= 2:
            _drain(k - 2)

        def _row(e, _):
            for kk in range(8):
                sl = pl.ds(kk * 16, 16)
                v = arows[e, sl] + brows[e, sl]
                mbuf[i][e, sl] = jnp.maximum(v, 0.0)
            return ()
        lax.fori_loop(0, CB, _row, (), unroll=False)

        pltpu.async_copy(mbuf[i], acc.at[idx_d], ssem[i], add=True)
        pltpu.async_copy(ones_v, cnt_s.at[idx_d], csem[i], add=True)

    def _drain(k):
        i = k % 2
        idx = dst_v.at[k]
        pltpu.make_async_copy(mbuf[i], acc.at[idx], ssem[i]).wait()
        pltpu.make_async_copy(ones_v, cnt_s.at[idx], csem[i]).wait()

    def _group(g, _):
        pltpu.sync_copy(dst_hbm.at[wid, pl.ds(g * GC, GC)], dst_v)
        pltpu.sync_copy(src_hbm.at[wid, pl.ds(g * GC, GC)], src_v)
        for k in range(GC):
            _chunk(k, GC)
        _drain(GC - 2)
        _drain(GC - 1)
        return ()
    lax.fori_loop(0, NCHUNK // GC, _group, (), unroll=False)

    ntail = NCHUNK - (NCHUNK // GC) * GC
    pltpu.sync_copy(dst_hbm.at[wid, pl.ds(NCHUNK - ntail, ntail)],
                    dst_v.at[pl.ds(0, ntail)])
    pltpu.sync_copy(src_hbm.at[wid, pl.ds(NCHUNK - ntail, ntail)],
                    src_v.at[pl.ds(0, ntail)])
    for k in range(ntail):
        _chunk(k, ntail)
    _drain(ntail - 2)
    _drain(ntail - 1)

    plsc.subcore_barrier()

    @pl.when(sid < 10)
    def _():
        for j in range(ROWS_PT // CB):
            base = sid * ROWS_PT + j * CB
            pltpu.sync_copy(acc.at[pl.ds(base, CB)], brows)
            pltpu.sync_copy(brows, p_hbm.at[cid, pl.ds(base, CB)])
        tail = sid * ROWS_PT + (ROWS_PT // CB) * CB
        pltpu.sync_copy(acc.at[pl.ds(tail, 40)], brows.at[pl.ds(0, 40)])
        pltpu.sync_copy(brows.at[pl.ds(0, 40)],
                        p_hbm.at[cid, pl.ds(tail, 40)])

        pltpu.sync_copy(cnt_s.at[pl.ds(sid * CNT_PT, CNT_PT)],
                        zcnt.at[pl.ds(0, CNT_PT)])

        @pl.when(cid == 0)
        def _():
            pltpu.sync_copy(zcnt.at[pl.ds(0, CNT_PT)],
                            cnt0_hbm.at[pl.ds(sid * CNT_PT, CNT_PT)])

        @pl.when(cid == 1)
        def _():
            pltpu.sync_copy(zcnt.at[pl.ds(0, CNT_PT)],
                            cnt1_hbm.at[pl.ds(sid * CNT_PT, CNT_PT)])


@functools.partial(
    pl.kernel,
    out_type=(
        jax.ShapeDtypeStruct((NC, N, D), jnp.float32),
        jax.ShapeDtypeStruct((N,), jnp.float32),
        jax.ShapeDtypeStruct((N,), jnp.float32),
    ),
    mesh=plsc.VectorSubcoreMesh(
        core_axis_name="c", subcore_axis_name="s", num_cores=NC, num_subcores=NS
    ),
    scratch_types=[
        pltpu.VMEM_SHARED((N, D), jnp.float32),
        pltpu.VMEM_SHARED((N,), jnp.float32),
        pltpu.VMEM((GC, CB), jnp.int32),
        pltpu.VMEM((GC, CB), jnp.int32),
        pltpu.VMEM((CB, D), jnp.float32),
        pltpu.VMEM((CB, D), jnp.float32),
        pltpu.VMEM((CB, D), jnp.float32),
        pltpu.VMEM((CB, D), jnp.float32),
        pltpu.VMEM((CB,), jnp.float32),
        pltpu.VMEM((1008,), jnp.float32),
        pltpu.SemaphoreType.DMA,
        pltpu.SemaphoreType.DMA,
        pltpu.SemaphoreType.DMA,
        pltpu.SemaphoreType.DMA,
        pltpu.SemaphoreType.DMA,
        pltpu.SemaphoreType.DMA,
    ],
)
def _edge_kernel(a_hbm, b_hbm, dst_hbm, src_hbm, p_hbm, cnt0_hbm, cnt1_hbm,
                 *scratch):
    _edge_body(a_hbm, b_hbm, dst_hbm, src_hbm, p_hbm, cnt0_hbm, cnt1_hbm,
               *scratch)


def _finalize_body(p_ref, c0_ref, c1_ref, o_ref):
    cnt = c0_ref[...] + c1_ref[...]
    inv = 1.0 / jnp.maximum(cnt, 1.0)
    o_ref[...] = (p_ref[0] + p_ref[1]) * inv[:, None]


def _finalize(p, cnt0, cnt1):
    return pl.pallas_call(
        _finalize_body,
        out_shape=jax.ShapeDtypeStruct((N, D), jnp.float32),
    )(p, cnt0, cnt1)


def kernel(x, edge_index, W, b):
    w1 = W[:, :D]
    w2 = W[:, D:]
    wa = (w1 - w2).T
    wb = w2.T
    b2d = b[None, :]
    a_nodes, b_nodes = _node_features(x, wa, wb, b2d)
    src = edge_index[0].reshape(NW, NCHUNK, CB)
    dst = edge_index[1].reshape(NW, NCHUNK, CB)
    p, cnt0, cnt1 = _edge_kernel(a_nodes, b_nodes, dst, src)
    return _finalize(p, cnt0, cnt1)
